# baseline (speedup 1.0000x reference)
import functools

import jax
import jax.numpy as jnp
from jax import lax
from jax.experimental import pallas as pl
from jax.experimental.pallas import tpu as pltpu
from jax.experimental.pallas import tpu_sc as plsc

N = 10000
D = 256
E = 160000
HALF = D // 2
NS = 16
K = 128
NCHUNK = 80
EPAD = NS * NCHUNK * K - E
NACC = N + 8
RPT = N // NS

_mesh = plsc.VectorSubcoreMesh(core_axis_name="c", subcore_axis_name="s")


@functools.partial(
    pl.kernel,
    out_type=(
        jax.ShapeDtypeStruct((N, HALF), jnp.bfloat16),
        jax.ShapeDtypeStruct((N, HALF), jnp.bfloat16),
    ),
    mesh=_mesh,
    scratch_types=[
        pltpu.VMEM((NCHUNK, K), jnp.int32),
        pltpu.VMEM((NCHUNK, K), jnp.int32),
        pltpu.VMEM((K, HALF), jnp.bfloat16),
        pltpu.VMEM((K, HALF), jnp.bfloat16),
        pltpu.VMEM((K, HALF), jnp.bfloat16),
        pltpu.VMEM((K, HALF), jnp.bfloat16),
        pltpu.VMEM_SHARED((NACC, HALF), jnp.bfloat16),
        pltpu.SemaphoreType.DMA,
        pltpu.SemaphoreType.DMA,
        pltpu.SemaphoreType.DMA,
        pltpu.SemaphoreType.DMA,
        pltpu.SemaphoreType.DMA,
        pltpu.SemaphoreType.DMA,
        pltpu.SemaphoreType.DMA,
        pltpu.SemaphoreType.DMA,
    ],
    compiler_params=pltpu.CompilerParams(use_tc_tiling_on_sc=False),
)
def _segsum(xw_lo, xw_hi, zeros_hbm, src_hbm, dst_hbm, out_lo, out_hi,
            src_v, dst_v, buf0, buf1, buf2, buf3, acc,
            g0, g1, g2, g3, s0, s1, s2, s3):
    c = lax.axis_index("c")
    s = lax.axis_index("s")
    bufs = (buf0, buf1, buf2, buf3)
    gsem = (g0, g1, g2, g3)
    ssem = (s0, s1, s2, s3)

    pltpu.sync_copy(src_hbm.at[s], src_v)
    pltpu.sync_copy(dst_hbm.at[s], dst_v)

    def gather(jc, slot):
        @pl.when(c == 0)
        def _():
            pltpu.async_copy(xw_lo.at[src_v.at[jc]], bufs[slot], gsem[slot])

        @pl.when(c == 1)
        def _():
            pltpu.async_copy(xw_hi.at[src_v.at[jc]], bufs[slot], gsem[slot])

    def scatter(jc, slot):
        pltpu.async_copy(bufs[slot], acc.at[dst_v.at[jc]], ssem[slot],
                         add=True)

    def drain(slot, sems):
        pltpu.make_async_copy(xw_lo.at[pl.ds(0, K)], bufs[slot],
                              sems[slot]).wait()

    gather(0, 0)
    gather(1, 1)
    gather(2, 2)
    rows = pl.ds(s * RPT, RPT)
    pltpu.sync_copy(zeros_hbm.at[rows], acc.at[rows])
    plsc.subcore_barrier()

    @pl.loop(0, NCHUNK, step=4)
    def _(j):
        for b in range(4):
            jc = j + b
            bn = (b + 3) % 4

            drain(b, gsem)
            scatter(jc, b)

            @pl.when(jc + 3 < NCHUNK)
            def _():
                @pl.when(jc >= 1)
                def _():
                    drain(bn, ssem)

                gather(jc + 3, bn)

    for b in range(4):
        drain(b, ssem)

    plsc.subcore_barrier()

    @pl.when(c == 0)
    def _():
        pltpu.sync_copy(acc.at[rows], out_lo.at[rows])

    @pl.when(c == 1)
    def _():
        pltpu.sync_copy(acc.at[rows], out_hi.at[rows])


def _mm_body(xlo_ref, xhi_ref, w_ref, b_ref, ylo_ref, yhi_ref, *, relu):
    xlo = xlo_ref[...].astype(jnp.float32)
    xhi = xhi_ref[...].astype(jnp.float32)
    if relu:
        xlo = jnp.maximum(xlo, 0.0)
        xhi = jnp.maximum(xhi, 0.0)
    y = (
        jnp.dot(xlo, w_ref[:HALF, :], preferred_element_type=jnp.float32)
        + jnp.dot(xhi, w_ref[HALF:, :], preferred_element_type=jnp.float32)
        + b_ref[...]
    )
    ylo_ref[...] = y[:, :HALF].astype(jnp.bfloat16)
    yhi_ref[...] = y[:, HALF:].astype(jnp.bfloat16)


_MM_ROWS = 1000


def _mm(xlo, xhi, W, b, relu):
    return pl.pallas_call(
        functools.partial(_mm_body, relu=relu),
        grid=(N // _MM_ROWS,),
        in_specs=[
            pl.BlockSpec((_MM_ROWS, HALF), lambda i: (i, 0)),
            pl.BlockSpec((_MM_ROWS, HALF), lambda i: (i, 0)),
            pl.BlockSpec((D, D), lambda i: (0, 0)),
            pl.BlockSpec((1, D), lambda i: (0, 0)),
        ],
        out_specs=[
            pl.BlockSpec((_MM_ROWS, HALF), lambda i: (i, 0)),
            pl.BlockSpec((_MM_ROWS, HALF), lambda i: (i, 0)),
        ],
        out_shape=[
            jax.ShapeDtypeStruct((N, HALF), jnp.bfloat16),
            jax.ShapeDtypeStruct((N, HALF), jnp.bfloat16),
        ],
    )(xlo, xhi, W, b.reshape(1, D))


def kernel(x, adj_t, edge_weight, W1, b1, W2, b2, W3, b3):
    src = jnp.concatenate(
        [adj_t[0].astype(jnp.int32), jnp.zeros((EPAD,), jnp.int32)]
    ).reshape(NS, NCHUNK, K)
    dst = jnp.concatenate(
        [adj_t[1].astype(jnp.int32), jnp.full((EPAD,), N, jnp.int32)]
    ).reshape(NS, NCHUNK, K)
    zeros = jnp.zeros((N, HALF), jnp.bfloat16)

    hlo, hhi = x[:, :HALF], x[:, HALF:]
    for W, b, relu in ((W1, b1, False), (W2, b2, True), (W3, b3, True)):
        ylo, yhi = _mm(hlo, hhi, W, b, relu)
        hlo, hhi = _segsum(ylo, yhi, zeros, src, dst)
    return jnp.concatenate([hlo, hhi], axis=1).astype(jnp.float32)


# speedup vs baseline: 1.0657x; 1.0657x over previous
import functools

import jax
import jax.numpy as jnp
from jax import lax
from jax.experimental import pallas as pl
from jax.experimental.pallas import tpu as pltpu
from jax.experimental.pallas import tpu_sc as plsc

N = 10000
D = 256
E = 160000
HALF = D // 2
NS = 16
K = 80
NCHUNK = 126
EPAD = NS * NCHUNK * K - E
NACC = N + 8
RPT = N // NS
SHIFT = 14

_mesh = plsc.VectorSubcoreMesh(core_axis_name="c", subcore_axis_name="s")


@functools.partial(
    pl.kernel,
    out_type=(
        jax.ShapeDtypeStruct((N, HALF), jnp.float32),
        jax.ShapeDtypeStruct((N, HALF), jnp.float32),
    ),
    mesh=_mesh,
    scratch_types=[
        pltpu.VMEM((NCHUNK, K), jnp.int32),
        pltpu.VMEM((3, K), jnp.int32),
        pltpu.VMEM((3, K), jnp.int32),
        pltpu.VMEM((K, HALF), jnp.float32),
        pltpu.VMEM((K, HALF), jnp.float32),
        pltpu.VMEM((K, HALF), jnp.float32),
        pltpu.VMEM_SHARED((NACC, HALF), jnp.float32),
        pltpu.SemaphoreType.DMA,
        pltpu.SemaphoreType.DMA,
        pltpu.SemaphoreType.DMA,
        pltpu.SemaphoreType.DMA,
        pltpu.SemaphoreType.DMA,
        pltpu.SemaphoreType.DMA,
    ],
    compiler_params=pltpu.CompilerParams(use_tc_tiling_on_sc=False),
)
def _segsum(xw_lo, xw_hi, zeros_hbm, packed_hbm, out_lo, out_hi,
            packed_v, srcr, dstr, buf0, buf1, buf2, acc,
            g0, g1, g2, s0, s1, s2):
    c = lax.axis_index("c")
    s = lax.axis_index("s")
    bufs = (buf0, buf1, buf2)
    gsem = (g0, g1, g2)
    ssem = (s0, s1, s2)

    pltpu.sync_copy(packed_hbm.at[s], packed_v)

    def unpack(jc, slot):
        for i in range(K // 16):
            p = packed_v[jc, pl.ds(i * 16, 16)]
            srcr[slot, pl.ds(i * 16, 16)] = lax.shift_right_logical(p, SHIFT)
            dstr[slot, pl.ds(i * 16, 16)] = lax.bitwise_and(
                p, jnp.int32((1 << SHIFT) - 1))

    def gather(slot):
        @pl.when(c == 0)
        def _():
            pltpu.async_copy(xw_lo.at[srcr.at[slot]], bufs[slot], gsem[slot])

        @pl.when(c == 1)
        def _():
            pltpu.async_copy(xw_hi.at[srcr.at[slot]], bufs[slot], gsem[slot])

    def scatter(slot):
        pltpu.async_copy(bufs[slot], acc.at[dstr.at[slot]], ssem[slot],
                         add=True)

    def drain(slot, sems):
        pltpu.make_async_copy(xw_lo.at[pl.ds(0, K)], bufs[slot],
                              sems[slot]).wait()

    unpack(0, 0)
    gather(0)
    unpack(1, 1)
    gather(1)
    rows = pl.ds(s * RPT, RPT)
    pltpu.sync_copy(zeros_hbm.at[rows], acc.at[rows])
    plsc.subcore_barrier()

    @pl.loop(0, NCHUNK, step=3)
    def _(j):
        for b in range(3):
            jc = j + b
            n = (b + 2) % 3

            drain(b, gsem)

            @pl.when(jc >= 1)
            def _():
                drain(n, ssem)

            @pl.when(jc + 2 < NCHUNK)
            def _():
                unpack(jc + 2, n)
                gather(n)

            scatter(b)

    drain((NCHUNK - 1) % 3, ssem)

    plsc.subcore_barrier()

    @pl.when(c == 0)
    def _():
        pltpu.sync_copy(acc.at[rows], out_lo.at[rows])

    @pl.when(c == 1)
    def _():
        pltpu.sync_copy(acc.at[rows], out_hi.at[rows])


def _mm_body(xlo_ref, xhi_ref, w_ref, b_ref, ylo_ref, yhi_ref, *, relu):
    xlo = xlo_ref[...]
    xhi = xhi_ref[...]
    if relu:
        xlo = jnp.maximum(xlo, 0.0)
        xhi = jnp.maximum(xhi, 0.0)
    y = (
        jnp.dot(xlo, w_ref[:HALF, :], preferred_element_type=jnp.float32)
        + jnp.dot(xhi, w_ref[HALF:, :], preferred_element_type=jnp.float32)
        + b_ref[...]
    )
    ylo_ref[...] = y[:, :HALF]
    yhi_ref[...] = y[:, HALF:]


_MM_ROWS = 1000


def _mm(xlo, xhi, W, b, relu):
    return pl.pallas_call(
        functools.partial(_mm_body, relu=relu),
        grid=(N // _MM_ROWS,),
        in_specs=[
            pl.BlockSpec((_MM_ROWS, HALF), lambda i: (i, 0)),
            pl.BlockSpec((_MM_ROWS, HALF), lambda i: (i, 0)),
            pl.BlockSpec((D, D), lambda i: (0, 0)),
            pl.BlockSpec((1, D), lambda i: (0, 0)),
        ],
        out_specs=[
            pl.BlockSpec((_MM_ROWS, HALF), lambda i: (i, 0)),
            pl.BlockSpec((_MM_ROWS, HALF), lambda i: (i, 0)),
        ],
        out_shape=[
            jax.ShapeDtypeStruct((N, HALF), jnp.float32),
            jax.ShapeDtypeStruct((N, HALF), jnp.float32),
        ],
    )(xlo, xhi, W, b.reshape(1, D))


def kernel(x, adj_t, edge_weight, W1, b1, W2, b2, W3, b3):
    src = adj_t[0].astype(jnp.int32)
    dst = adj_t[1].astype(jnp.int32)
    packed = jnp.concatenate(
        [(src << SHIFT) | dst, jnp.full((EPAD,), N, jnp.int32)]
    ).reshape(NS, NCHUNK, K)
    zeros = jnp.zeros((N, HALF), jnp.float32)

    hlo, hhi = x[:, :HALF], x[:, HALF:]
    for W, b, relu in ((W1, b1, False), (W2, b2, True), (W3, b3, True)):
        ylo, yhi = _mm(hlo, hhi, W, b, relu)
        hlo, hhi = _segsum(ylo, yhi, zeros, packed)
    return jnp.concatenate([hlo, hhi], axis=1)


# speedup vs baseline: 1.2529x; 1.1756x over previous
import functools

import jax
import jax.numpy as jnp
from jax import lax
from jax.experimental import pallas as pl
from jax.experimental.pallas import tpu as pltpu
from jax.experimental.pallas import tpu_sc as plsc

N = 10000
D = 256
E = 160000
HALF = D // 2
NS = 16
EPT = E // NS
K = 100
NCHUNK = EPT // K
RPT = N // NS

_mesh = plsc.VectorSubcoreMesh(core_axis_name="c", subcore_axis_name="s")


@functools.partial(
    pl.kernel,
    out_type=(
        jax.ShapeDtypeStruct((N, HALF), jnp.float32),
        jax.ShapeDtypeStruct((N, HALF), jnp.float32),
    ),
    mesh=_mesh,
    scratch_types=[
        pltpu.VMEM((NCHUNK, K), jnp.int32),
        pltpu.VMEM((NCHUNK, K), jnp.int32),
        pltpu.VMEM((K, HALF), jnp.float32),
        pltpu.VMEM((K, HALF), jnp.float32),
        pltpu.VMEM_SHARED((N, HALF), jnp.float32),
        pltpu.SemaphoreType.DMA,
        pltpu.SemaphoreType.DMA,
        pltpu.SemaphoreType.DMA,
        pltpu.SemaphoreType.DMA,
    ],
    compiler_params=pltpu.CompilerParams(use_tc_tiling_on_sc=False),
)
def _segsum(xw_lo, xw_hi, zeros_hbm, src_hbm, dst_hbm, out_lo, out_hi,
            src_v, dst_v, buf0, buf1, acc, g0, g1, s0, s1):
    c = lax.axis_index("c")
    s = lax.axis_index("s")

    pltpu.sync_copy(src_hbm.at[s], src_v)
    pltpu.sync_copy(dst_hbm.at[s], dst_v)

    def gather(j, buf, sem):
        @pl.when(c == 0)
        def _():
            pltpu.async_copy(xw_lo.at[src_v.at[j]], buf, sem)

        @pl.when(c == 1)
        def _():
            pltpu.async_copy(xw_hi.at[src_v.at[j]], buf, sem)

    def scatter(j, buf, sem):
        pltpu.async_copy(buf, acc.at[dst_v.at[j]], sem, add=True)

    def drain(buf, sem):
        pltpu.make_async_copy(xw_lo.at[pl.ds(0, K)], buf, sem).wait()

    gather(0, buf0, g0)
    rows = pl.ds(s * RPT, RPT)
    pltpu.sync_copy(zeros_hbm.at[rows], acc.at[rows])
    plsc.subcore_barrier()

    @pl.loop(0, NCHUNK, step=2)
    def _(j):
        drain(buf0, g0)

        @pl.when(j >= 1)
        def _():
            drain(buf1, s1)

        gather(j + 1, buf1, g1)
        scatter(j, buf0, s0)
        drain(buf1, g1)

        @pl.when(j + 2 < NCHUNK)
        def _():
            drain(buf0, s0)
            gather(j + 2, buf0, g0)

        scatter(j + 1, buf1, s1)

    drain(buf0, s0)
    drain(buf1, s1)

    plsc.subcore_barrier()

    @pl.when(c == 0)
    def _():
        pltpu.sync_copy(acc.at[rows], out_lo.at[rows])

    @pl.when(c == 1)
    def _():
        pltpu.sync_copy(acc.at[rows], out_hi.at[rows])


def _mm_body(xlo_ref, xhi_ref, w_ref, b_ref, ylo_ref, yhi_ref, *, relu):
    xlo = xlo_ref[...]
    xhi = xhi_ref[...]
    if relu:
        xlo = jnp.maximum(xlo, 0.0)
        xhi = jnp.maximum(xhi, 0.0)
    y = (
        jnp.dot(xlo, w_ref[:HALF, :], preferred_element_type=jnp.float32)
        + jnp.dot(xhi, w_ref[HALF:, :], preferred_element_type=jnp.float32)
        + b_ref[...]
    )
    ylo_ref[...] = y[:, :HALF]
    yhi_ref[...] = y[:, HALF:]


_MM_ROWS = 1000


def _mm(xlo, xhi, W, b, relu):
    return pl.pallas_call(
        functools.partial(_mm_body, relu=relu),
        grid=(N // _MM_ROWS,),
        in_specs=[
            pl.BlockSpec((_MM_ROWS, HALF), lambda i: (i, 0)),
            pl.BlockSpec((_MM_ROWS, HALF), lambda i: (i, 0)),
            pl.BlockSpec((D, D), lambda i: (0, 0)),
            pl.BlockSpec((1, D), lambda i: (0, 0)),
        ],
        out_specs=[
            pl.BlockSpec((_MM_ROWS, HALF), lambda i: (i, 0)),
            pl.BlockSpec((_MM_ROWS, HALF), lambda i: (i, 0)),
        ],
        out_shape=[
            jax.ShapeDtypeStruct((N, HALF), jnp.float32),
            jax.ShapeDtypeStruct((N, HALF), jnp.float32),
        ],
    )(xlo, xhi, W, b.reshape(1, D))


def kernel(x, adj_t, edge_weight, W1, b1, W2, b2, W3, b3):
    src = adj_t[0].astype(jnp.int32).reshape(NS, NCHUNK, K)
    dst = adj_t[1].astype(jnp.int32).reshape(NS, NCHUNK, K)
    zeros = jnp.zeros((N, HALF), jnp.float32)

    hlo, hhi = x[:, :HALF], x[:, HALF:]
    for W, b, relu in ((W1, b1, False), (W2, b2, True), (W3, b3, True)):
        ylo, yhi = _mm(hlo, hhi, W, b, relu)
        hlo, hhi = _segsum(ylo, yhi, zeros, src, dst)
    return jnp.concatenate([hlo, hhi], axis=1)


# speedup vs baseline: 1.7105x; 1.3653x over previous
import functools

import jax
import jax.numpy as jnp
from jax import lax
from jax.experimental import pallas as pl
from jax.experimental.pallas import tpu as pltpu
from jax.experimental.pallas import tpu_sc as plsc

N = 10000
D = 256
E = 160000
HALF = D // 2
NS = 16
EPT = E // NS
K = 100
NHALVES = 2
HCHUNK = 50
RPT = N // NS

_mesh = plsc.VectorSubcoreMesh(core_axis_name="c", subcore_axis_name="s")


@functools.partial(
    pl.kernel,
    out_type=(
        jax.ShapeDtypeStruct((N, HALF), jnp.float32),
        jax.ShapeDtypeStruct((N, HALF), jnp.float32),
    ),
    mesh=_mesh,
    scratch_types=[
        pltpu.VMEM((HCHUNK, K), jnp.int32),
        pltpu.VMEM((HCHUNK, K), jnp.int32),
        pltpu.VMEM((K, HALF), jnp.float32),
        pltpu.VMEM((K, HALF), jnp.float32),
        pltpu.VMEM((K, HALF), jnp.float32),
        pltpu.VMEM_SHARED((N, HALF), jnp.float32),
        pltpu.SemaphoreType.DMA,
        pltpu.SemaphoreType.DMA,
        pltpu.SemaphoreType.DMA,
        pltpu.SemaphoreType.DMA,
        pltpu.SemaphoreType.DMA,
        pltpu.SemaphoreType.DMA,
    ],
    compiler_params=pltpu.CompilerParams(use_tc_tiling_on_sc=False),
)
def _segsum(xw_lo, xw_hi, zeros_hbm, src_hbm, dst_hbm, out_lo, out_hi,
            src_v, dst_v, buf0, buf1, buf2, acc, g0, g1, g2, s0, s1, s2):
    c = lax.axis_index("c")
    s = lax.axis_index("s")
    bufs = (buf0, buf1, buf2)
    gsem = (g0, g1, g2)
    ssem = (s0, s1, s2)

    def gather(q, slot):
        @pl.when(c == 0)
        def _():
            pltpu.async_copy(xw_lo.at[src_v.at[q]], bufs[slot], gsem[slot])

        @pl.when(c == 1)
        def _():
            pltpu.async_copy(xw_hi.at[src_v.at[q]], bufs[slot], gsem[slot])

    def scatter(q, slot):
        pltpu.async_copy(bufs[slot], acc.at[dst_v.at[q]], ssem[slot],
                         add=True)

    def drain(slot, sems):
        pltpu.make_async_copy(xw_lo.at[pl.ds(0, K)], bufs[slot],
                              sems[slot]).wait()

    rows = pl.ds(s * RPT, RPT)

    for h in range(NHALVES):
        pltpu.sync_copy(src_hbm.at[s, h], src_v)
        pltpu.sync_copy(dst_hbm.at[s, h], dst_v)

        gather(0, 0)
        gather(1, 1)
        if h == 0:
            pltpu.sync_copy(zeros_hbm.at[rows], acc.at[rows])
            plsc.subcore_barrier()

        def visit(q, b, n, tail):
            drain(b, gsem)
            if tail:
                drain(n, ssem)
            else:
                @pl.when(q >= 1)
                def _():
                    drain(n, ssem)

                gather(q + 2, n)
            scatter(q, b)

        @pl.loop(0, HCHUNK - 2, step=3)
        def _(j):
            for b in range(3):
                visit(j + b, b, (b + 2) % 3, False)

        visit(HCHUNK - 2, (HCHUNK - 2) % 3, HCHUNK % 3, True)
        visit(HCHUNK - 1, (HCHUNK - 1) % 3, (HCHUNK + 1) % 3, True)
        drain((HCHUNK - 1) % 3, ssem)

    plsc.subcore_barrier()

    @pl.when(c == 0)
    def _():
        pltpu.sync_copy(acc.at[rows], out_lo.at[rows])

    @pl.when(c == 1)
    def _():
        pltpu.sync_copy(acc.at[rows], out_hi.at[rows])


def _mm_body(xlo_ref, xhi_ref, w_ref, b_ref, ylo_ref, yhi_ref, *, relu):
    xlo = xlo_ref[...]
    xhi = xhi_ref[...]
    if relu:
        xlo = jnp.maximum(xlo, 0.0)
        xhi = jnp.maximum(xhi, 0.0)
    y = (
        jnp.dot(xlo, w_ref[:HALF, :], preferred_element_type=jnp.float32)
        + jnp.dot(xhi, w_ref[HALF:, :], preferred_element_type=jnp.float32)
        + b_ref[...]
    )
    ylo_ref[...] = y[:, :HALF]
    yhi_ref[...] = y[:, HALF:]


_MM_ROWS = 1000


def _mm(xlo, xhi, W, b, relu):
    return pl.pallas_call(
        functools.partial(_mm_body, relu=relu),
        grid=(N // _MM_ROWS,),
        in_specs=[
            pl.BlockSpec((_MM_ROWS, HALF), lambda i: (i, 0)),
            pl.BlockSpec((_MM_ROWS, HALF), lambda i: (i, 0)),
            pl.BlockSpec((D, D), lambda i: (0, 0)),
            pl.BlockSpec((1, D), lambda i: (0, 0)),
        ],
        out_specs=[
            pl.BlockSpec((_MM_ROWS, HALF), lambda i: (i, 0)),
            pl.BlockSpec((_MM_ROWS, HALF), lambda i: (i, 0)),
        ],
        out_shape=[
            jax.ShapeDtypeStruct((N, HALF), jnp.float32),
            jax.ShapeDtypeStruct((N, HALF), jnp.float32),
        ],
    )(xlo, xhi, W, b.reshape(1, D))


def kernel(x, adj_t, edge_weight, W1, b1, W2, b2, W3, b3):
    src = adj_t[0].astype(jnp.int32).reshape(NS, NHALVES, HCHUNK, K)
    dst = adj_t[1].astype(jnp.int32).reshape(NS, NHALVES, HCHUNK, K)
    zeros = jnp.zeros((N, HALF), jnp.float32)

    hlo, hhi = x[:, :HALF], x[:, HALF:]
    for W, b, relu in ((W1, b1, False), (W2, b2, True), (W3, b3, True)):
        ylo, yhi = _mm(hlo, hhi, W, b, relu)
        hlo, hhi = _segsum(ylo, yhi, zeros, src, dst)
    return jnp.concatenate([hlo, hhi], axis=1)
